# baseline (speedup 1.0000x reference)
import functools

import jax
import jax.numpy as jnp
from jax import lax
from jax.experimental import pallas as pl
from jax.experimental.pallas import tpu as pltpu

N_PAD_BLK = 1024

GATLIKE = ['gat', 'gat_sym', 'gat_linear', 'geniepath']


def _weight_prep_body(wcos_ref, alcos_ref, arcos_ref, wgen_ref, algen_ref,
                      argen_ref, wsc_ref, avec_ref, wbig_ref):
    wsc = wsc_ref[...]
    av = avec_ref[...]
    cols = []
    for p in range(4):
        cols.append(jnp.dot(wsc[p], av[p, 0][:, None],
                            preferred_element_type=jnp.float32))
        cols.append(jnp.dot(wsc[p], av[p, 1][:, None],
                            preferred_element_type=jnp.float32))
    sc_cols = jnp.concatenate(cols, axis=1)
    wcos = wcos_ref[...]
    mcos = jnp.dot(wcos * (alcos_ref[...] * arcos_ref[...])[None, :],
                   wcos.T, preferred_element_type=jnp.float32)
    wgen = wgen_ref[...]
    ucols = wgen * algen_ref[...][None, :]
    vcols = wgen * argen_ref[...][None, :]
    pad = jnp.zeros((256, 120), dtype=jnp.float32)
    wbig_ref[...] = jnp.concatenate([sc_cols, mcos, ucols, vcols, pad], axis=1)


def _weight_prep(p):
    wsc = jnp.stack([p[n]['W'] for n in GATLIKE])
    avec = jnp.stack([jnp.stack([p[n]['al'], p[n]['ar']]) for n in GATLIKE])
    return pl.pallas_call(
        _weight_prep_body,
        out_shape=jax.ShapeDtypeStruct((256, 896), jnp.float32),
    )(p['gat_cos']['W'], p['gat_cos']['al'], p['gat_cos']['ar'],
      p['gat_generalized_linear']['W'], p['gat_generalized_linear']['al'],
      p['gat_generalized_linear']['ar'], wsc, avec)


def _node_prep_body(x_ref, wbig_ref, out_ref):
    out_ref[...] = jnp.dot(x_ref[...], wbig_ref[...],
                           preferred_element_type=jnp.float32)


def _node_prep(xp, wbig):
    npad = xp.shape[0]
    return pl.pallas_call(
        _node_prep_body,
        grid=(npad // N_PAD_BLK,),
        in_specs=[pl.BlockSpec((N_PAD_BLK, 256), lambda i: (i, 0)),
                  pl.BlockSpec((256, 896), lambda i: (0, 0))],
        out_specs=pl.BlockSpec((N_PAD_BLK, 896), lambda i: (i, 0)),
        out_shape=jax.ShapeDtypeStruct((npad, 896), jnp.float32),
    )(xp, wbig)


def _final_body(x_ref, a1_ref, amax_ref, agcn_ref, aatt_ref, invdeg_ref,
                dinv2_ref, wmix_ref, wl_ref, wr_ref, bs_ref, wgcn_ref,
                bgcn_ref, w1_ref, b1_ref, w2_ref, b2_ref, eps_ref, wgat_ref,
                bgat_ref, wgen_ref, wlstm_ref, blstm_ref, out_ref):
    x = x_ref[...]
    a1 = a1_ref[...]
    invdeg = invdeg_ref[...]
    wmix = wmix_ref[...]

    def elu(v):
        return jnp.where(v > 0, v, jnp.exp(jnp.minimum(v, 0.0)) - 1.0)

    def mm(a, w):
        return jnp.dot(a, w, preferred_element_type=jnp.float32)

    acc = jnp.zeros_like(x)
    aggs = [a1 * invdeg, a1, amax_ref[...]]
    for k in range(3):
        out = mm(aggs[k], wl_ref[k]) + mm(x, wr_ref[k]) + bs_ref[k][None, :]
        acc += wmix[0, k] * elu(out)
    out = mm(agcn_ref[...] + dinv2_ref[...] * x, wgcn_ref[...]) \
        + bgcn_ref[...][None, :]
    acc += wmix[0, 3] * elu(out)
    h = (1.0 + eps_ref[0, 0]) * x + a1
    h = jnp.maximum(mm(h, w1_ref[...]) + b1_ref[...][None, :], 0.0)
    out = mm(h, w2_ref[...]) + b2_ref[...][None, :]
    acc += wmix[0, 4] * elu(out)
    for k in range(5):
        out = mm(aatt_ref[k], wgat_ref[k]) + bgat_ref[k][None, :]
        acc += wmix[0, 5 + k] * elu(out)
    hb = jnp.tanh(mm(aatt_ref[5], wgen_ref[...]))
    gates = mm(hb, wlstm_ref[...]) + blstm_ref[...][None, :]
    i = gates[:, 0:256]
    g = gates[:, 512:768]
    o = gates[:, 768:1024]
    c = jax.nn.sigmoid(i) * jnp.tanh(g)
    out = jax.nn.sigmoid(o) * jnp.tanh(c)
    acc += wmix[0, 10] * elu(out)
    out_ref[...] = acc


def _final(xp, a1, amax, agcn, aatt, invdeg, dinv2, weights, p):
    npad = xp.shape[0]
    nb = npad // N_PAD_BLK
    wl = jnp.stack([p[n]['Wl'] for n in ['sage', 'sage_sum', 'sage_max']])
    wr = jnp.stack([p[n]['Wr'] for n in ['sage', 'sage_sum', 'sage_max']])
    bs = jnp.stack([p[n]['b'] for n in ['sage', 'sage_sum', 'sage_max']])
    gat_names = ['gat', 'gat_sym', 'gat_cos', 'gat_linear',
                 'gat_generalized_linear']
    wgat = jnp.stack([p[n]['W'] for n in gat_names])
    bgat = jnp.stack([p[n]['b'] for n in gat_names])
    wmix = weights.reshape(1, 11)
    eps = p['gin']['eps'].reshape(1, 1)

    full = lambda *s: pl.BlockSpec(s, lambda i: (0,) * len(s))
    blk = pl.BlockSpec((N_PAD_BLK, 256), lambda i: (i, 0))
    blk1 = pl.BlockSpec((N_PAD_BLK, 1), lambda i: (i, 0))
    in_specs = [
        blk, blk,
        blk, blk,
        pl.BlockSpec((6, N_PAD_BLK, 256), lambda i: (0, i, 0)),
        blk1, blk1,
        full(1, 11),
        full(3, 256, 256), full(3, 256, 256), full(3, 256),
        full(256, 256), full(256,),
        full(256, 256), full(256,), full(256, 256), full(256,), full(1, 1),
        full(5, 256, 256), full(5, 256),
        full(256, 256), full(256, 1024), full(1024,),
    ]
    return pl.pallas_call(
        _final_body,
        grid=(nb,),
        in_specs=in_specs,
        out_specs=blk,
        out_shape=jax.ShapeDtypeStruct((npad, 256), jnp.float32),
    )(xp, a1, amax, agcn, aatt, invdeg, dinv2, wmix, wl, wr, bs,
      p['gcn']['W'], p['gcn']['b'], p['gin']['W1'], p['gin']['b1'],
      p['gin']['W2'], p['gin']['b2'], eps, wgat, bgat,
      p['geniepath']['W'], p['geniepath']['Wlstm'], p['geniepath']['blstm'])


def kernel(x, weights, edge_index, params):
    n = x.shape[0]
    npad = ((n + N_PAD_BLK - 1) // N_PAD_BLK) * N_PAD_BLK
    xp = jnp.pad(x, ((0, npad - n), (0, 0)))
    src = edge_index[0]
    dst = edge_index[1]

    wbig = _weight_prep(params)
    nodef = _node_prep(xp, wbig)

    e = src.shape[0]
    ones = jnp.ones((e,), jnp.float32)
    deg = jax.ops.segment_sum(ones, dst, num_segments=npad)
    invdeg = (1.0 / jnp.clip(deg, 1.0))[:, None]
    dinv = 1.0 / jnp.sqrt(deg + 1.0)
    dinv2 = (dinv * dinv)[:, None]

    sc = nodef[:, :8]
    G = nodef[:, 8:264]
    U = nodef[:, 264:520]
    V = nodef[:, 520:776]
    ag = params['gat_generalized_linear']['ag']

    lrelu = lambda v: jnp.where(v > 0, v, 0.2 * v)
    lo = jnp.stack([
        lrelu(sc[src, 0] + sc[dst, 1]),
        lrelu(sc[src, 2] + sc[dst, 3]) + lrelu(sc[dst, 2] + sc[src, 3]),
        jnp.sum(G[src] * xp[dst], axis=-1),
        jnp.tanh(sc[src, 4] + sc[dst, 5]),
        jnp.tanh(U[src] + V[dst]) @ ag,
        lrelu(sc[src, 6] + sc[dst, 7]),
    ])

    def seg_softmax(logits):
        m = jax.ops.segment_max(logits, dst, num_segments=npad)
        m = jnp.where(jnp.isfinite(m), m, 0.0)
        ee = jnp.exp(logits - m[dst])
        den = jax.ops.segment_sum(ee, dst, num_segments=npad)
        return ee / (den[dst] + 1e-16)

    att = jax.vmap(seg_softmax)(lo)
    norm = dinv[src] * dinv[dst]

    xs = xp[src]
    a1 = jax.ops.segment_sum(xs, dst, num_segments=npad)
    amax = jax.ops.segment_max(xs, dst, num_segments=npad)
    amax = jnp.where(jnp.isfinite(amax), amax, 0.0)
    agcn = jax.ops.segment_sum(xs * norm[:, None], dst, num_segments=npad)
    aatt = jax.vmap(
        lambda a: jax.ops.segment_sum(xs * a[:, None], dst,
                                      num_segments=npad))(att)

    res = _final(xp, a1, amax, agcn, aatt, invdeg, dinv2, weights, params)
    return res[:n]


# speedup vs baseline: 1.0279x; 1.0279x over previous
import functools

import jax
import jax.numpy as jnp
from jax import lax
from jax.experimental import pallas as pl
from jax.experimental.pallas import tpu as pltpu
from jax.experimental.pallas import tpu_sc as plsc

N_PAD_BLK = 1024

GATLIKE = ['gat', 'gat_sym', 'gat_linear', 'geniepath']


def _weight_prep_body(wcos_ref, alcos_ref, arcos_ref, wgen_ref, algen_ref,
                      argen_ref, wsc_ref, avec_ref, wbig_ref):
    wsc = wsc_ref[...]
    av = avec_ref[...]
    cols = []
    for p in range(4):
        cols.append(jnp.dot(wsc[p], av[p, 0][:, None],
                            preferred_element_type=jnp.float32))
        cols.append(jnp.dot(wsc[p], av[p, 1][:, None],
                            preferred_element_type=jnp.float32))
    sc_cols = jnp.concatenate(cols, axis=1)
    wcos = wcos_ref[...]
    mcos = jnp.dot(wcos * (alcos_ref[...] * arcos_ref[...])[None, :],
                   wcos.T, preferred_element_type=jnp.float32)
    wgen = wgen_ref[...]
    ucols = wgen * algen_ref[...][None, :]
    vcols = wgen * argen_ref[...][None, :]
    pad = jnp.zeros((256, 120), dtype=jnp.float32)
    wbig_ref[...] = jnp.concatenate([sc_cols, mcos, ucols, vcols, pad], axis=1)


def _weight_prep(p):
    wsc = jnp.stack([p[n]['W'] for n in GATLIKE])
    avec = jnp.stack([jnp.stack([p[n]['al'], p[n]['ar']]) for n in GATLIKE])
    return pl.pallas_call(
        _weight_prep_body,
        out_shape=jax.ShapeDtypeStruct((256, 896), jnp.float32),
    )(p['gat_cos']['W'], p['gat_cos']['al'], p['gat_cos']['ar'],
      p['gat_generalized_linear']['W'], p['gat_generalized_linear']['al'],
      p['gat_generalized_linear']['ar'], wsc, avec)


def _node_prep_body(x_ref, wbig_ref, sc_ref, g_ref, u_ref, v_ref):
    h = jnp.dot(x_ref[...], wbig_ref[...], preferred_element_type=jnp.float32)
    sc_ref[...] = h[:, 0:8]
    g_ref[...] = h[:, 8:264]
    u_ref[...] = h[:, 264:520]
    v_ref[...] = h[:, 520:776]


def _node_prep(xp, wbig):
    npad = xp.shape[0]
    blk = lambda w: pl.BlockSpec((N_PAD_BLK, w), lambda i: (i, 0))
    return pl.pallas_call(
        _node_prep_body,
        grid=(npad // N_PAD_BLK,),
        in_specs=[pl.BlockSpec((N_PAD_BLK, 256), lambda i: (i, 0)),
                  pl.BlockSpec((256, 896), lambda i: (0, 0))],
        out_specs=[blk(8), blk(256), blk(256), blk(256)],
        out_shape=[jax.ShapeDtypeStruct((npad, 8), jnp.float32),
                   jax.ShapeDtypeStruct((npad, 256), jnp.float32),
                   jax.ShapeDtypeStruct((npad, 256), jnp.float32),
                   jax.ShapeDtypeStruct((npad, 256), jnp.float32)],
    )(xp, wbig)


def _final_body(x_ref, a1_ref, amax_ref, agcn_ref, aatt_ref, invdeg_ref,
                dinv2_ref, wmix_ref, wl_ref, wr_ref, bs_ref, wgcn_ref,
                bgcn_ref, w1_ref, b1_ref, w2_ref, b2_ref, eps_ref, wgat_ref,
                bgat_ref, wgen_ref, wlstm_ref, blstm_ref, out_ref):
    x = x_ref[...]
    a1 = a1_ref[...]
    invdeg = invdeg_ref[...]
    wmix = wmix_ref[...]

    def elu(v):
        return jnp.where(v > 0, v, jnp.exp(jnp.minimum(v, 0.0)) - 1.0)

    def mm(a, w):
        return jnp.dot(a, w, preferred_element_type=jnp.float32)

    acc = jnp.zeros_like(x)
    aggs = [a1 * invdeg, a1, amax_ref[...]]
    for k in range(3):
        out = mm(aggs[k], wl_ref[k]) + mm(x, wr_ref[k]) + bs_ref[k][None, :]
        acc += wmix[0, k] * elu(out)
    out = mm(agcn_ref[...] + dinv2_ref[...] * x, wgcn_ref[...]) \
        + bgcn_ref[...][None, :]
    acc += wmix[0, 3] * elu(out)
    h = (1.0 + eps_ref[0, 0]) * x + a1
    h = jnp.maximum(mm(h, w1_ref[...]) + b1_ref[...][None, :], 0.0)
    out = mm(h, w2_ref[...]) + b2_ref[...][None, :]
    acc += wmix[0, 4] * elu(out)
    for k in range(5):
        out = mm(aatt_ref[k], wgat_ref[k]) + bgat_ref[k][None, :]
        acc += wmix[0, 5 + k] * elu(out)
    hb = jnp.tanh(mm(aatt_ref[5], wgen_ref[...]))
    gates = mm(hb, wlstm_ref[...]) + blstm_ref[...][None, :]
    i = gates[:, 0:256]
    g = gates[:, 512:768]
    o = gates[:, 768:1024]
    c = jax.nn.sigmoid(i) * jnp.tanh(g)
    out = jax.nn.sigmoid(o) * jnp.tanh(c)
    acc += wmix[0, 10] * elu(out)
    out_ref[...] = acc


def _final(xp, a1, amax, agcn, aatt, invdeg, dinv2, weights, p):
    npad = xp.shape[0]
    nb = npad // N_PAD_BLK
    wl = jnp.stack([p[n]['Wl'] for n in ['sage', 'sage_sum', 'sage_max']])
    wr = jnp.stack([p[n]['Wr'] for n in ['sage', 'sage_sum', 'sage_max']])
    bs = jnp.stack([p[n]['b'] for n in ['sage', 'sage_sum', 'sage_max']])
    gat_names = ['gat', 'gat_sym', 'gat_cos', 'gat_linear',
                 'gat_generalized_linear']
    wgat = jnp.stack([p[n]['W'] for n in gat_names])
    bgat = jnp.stack([p[n]['b'] for n in gat_names])
    wmix = weights.reshape(1, 11)
    eps = p['gin']['eps'].reshape(1, 1)

    full = lambda *s: pl.BlockSpec(s, lambda i: (0,) * len(s))
    blk = pl.BlockSpec((N_PAD_BLK, 256), lambda i: (i, 0))
    blk1 = pl.BlockSpec((N_PAD_BLK, 1), lambda i: (i, 0))
    in_specs = [
        blk, blk,
        blk, blk,
        pl.BlockSpec((6, N_PAD_BLK, 256), lambda i: (0, i, 0)),
        blk1, blk1,
        full(1, 11),
        full(3, 256, 256), full(3, 256, 256), full(3, 256),
        full(256, 256), full(256,),
        full(256, 256), full(256,), full(256, 256), full(256,), full(1, 1),
        full(5, 256, 256), full(5, 256),
        full(256, 256), full(256, 1024), full(1024,),
    ]
    return pl.pallas_call(
        _final_body,
        grid=(nb,),
        in_specs=in_specs,
        out_specs=blk,
        out_shape=jax.ShapeDtypeStruct((npad, 256), jnp.float32),
    )(xp, a1, amax, agcn, aatt, invdeg, dinv2, wmix, wl, wr, bs,
      p['gcn']['W'], p['gcn']['b'], p['gin']['W1'], p['gin']['b1'],
      p['gin']['W2'], p['gin']['b2'], eps, wgat, bgat,
      p['geniepath']['W'], p['geniepath']['Wlstm'], p['geniepath']['blstm'])


SC_NC = 2
SC_NS = 16
SC_NW = SC_NC * SC_NS
K1_C = 32


def _k1_body(srcp, dstp, sc8f, gmat, xmat, umat, vmat, agvec, lo_out,
             src_v, dst_v, tab_v, g_v, x_v, u_v, v_v, ag_v, lo_v,
             part_v, sem):
    epw = srcp.shape[0] // SC_NW
    npad8 = sc8f.shape[0]
    wid = lax.axis_index("s") * SC_NC + lax.axis_index("c")
    base_w = wid * epw
    pltpu.sync_copy(agvec, ag_v)
    pltpu.sync_copy(sc8f, tab_v)

    def lrelu(z):
        return jnp.where(z > 0, z, 0.2 * z)

    def tanh16(z):
        e2 = jnp.exp(z + z)
        return 1.0 - 2.0 / (e2 + 1.0)

    def chunk(i, carry):
        base = base_w + i * K1_C
        pltpu.sync_copy(srcp.at[pl.ds(base, K1_C)], src_v)
        pltpu.sync_copy(dstp.at[pl.ds(base, K1_C)], dst_v)
        pltpu.async_copy(gmat.at[src_v], g_v, sem).wait()
        pltpu.async_copy(xmat.at[dst_v], x_v, sem).wait()
        pltpu.async_copy(umat.at[src_v], u_v, sem).wait()
        pltpu.async_copy(vmat.at[dst_v], v_v, sem).wait()

        def edge(e, carry2):
            acc_c = jnp.zeros((16,), jnp.float32)
            acc_g = jnp.zeros((16,), jnp.float32)
            for k in range(16):
                ks = pl.ds(k * 16, 16)
                acc_c += g_v[e, ks] * x_v[e, ks]
                z = u_v[e, ks] + v_v[e, ks]
                acc_g += tanh16(z) * ag_v[ks]
            part_v[pl.ds(e * 16, 16)] = acc_c
            part_v[pl.ds(K1_C * 16 + e * 16, 16)] = acc_g
            return carry2

        lax.fori_loop(0, K1_C, edge, 0, unroll=False)

        for g16 in range(K1_C // 16):
            rows = lax.iota(jnp.int32, 16) + g16 * 16
            s16 = src_v[pl.ds(g16 * 16, 16)]
            d16 = dst_v[pl.ds(g16 * 16, 16)]

            def sf(n16, f):
                return plsc.load_gather(tab_v, [n16 * 8 + f])

            lo_gat = lrelu(sf(s16, 0) + sf(d16, 1))
            lo_sym = (lrelu(sf(s16, 2) + sf(d16, 3))
                      + lrelu(sf(d16, 2) + sf(s16, 3)))
            lo_lin = tanh16(sf(s16, 4) + sf(d16, 5))
            lo_gen8 = lrelu(sf(s16, 6) + sf(d16, 7))
            lo_cos = jnp.zeros((16,), jnp.float32)
            lo_gnl = jnp.zeros((16,), jnp.float32)
            for k in range(16):
                lo_cos += plsc.load_gather(part_v, [rows * 16 + k])
                lo_gnl += plsc.load_gather(part_v, [K1_C * 16 + rows * 16 + k])
            for f, val in ((0, lo_gat), (1, lo_sym), (2, lo_cos),
                           (3, lo_lin), (4, lo_gnl), (5, lo_gen8)):
                plsc.store_scatter(lo_v, [rows * 8 + f], val)
        pltpu.sync_copy(lo_v, lo_out.at[pl.ds(base * 8, K1_C * 8)])
        return carry

    lax.fori_loop(0, epw // K1_C, chunk, 0, unroll=False)


def _k1_logits(srcp, dstp, sc8, gmat, xmat, umat, vmat, ag):
    epad = srcp.shape[0]
    npad = sc8.shape[0]
    mesh = plsc.VectorSubcoreMesh(core_axis_name="c", subcore_axis_name="s")
    f = pl.kernel(
        _k1_body,
        out_type=jax.ShapeDtypeStruct((epad * 8,), jnp.float32),
        mesh=mesh,
        compiler_params=pltpu.CompilerParams(needs_layout_passes=False),
        scratch_types=[
            pltpu.VMEM((K1_C,), jnp.int32),
            pltpu.VMEM((K1_C,), jnp.int32),
            pltpu.VMEM((npad * 8,), jnp.float32),
            pltpu.VMEM((K1_C, 256), jnp.float32),
            pltpu.VMEM((K1_C, 256), jnp.float32),
            pltpu.VMEM((K1_C, 256), jnp.float32),
            pltpu.VMEM((K1_C, 256), jnp.float32),
            pltpu.VMEM((256,), jnp.float32),
            pltpu.VMEM((K1_C * 8,), jnp.float32),
            pltpu.VMEM((2 * K1_C * 16,), jnp.float32),
            pltpu.SemaphoreType.DMA,
        ],
    )
    return f(srcp, dstp, sc8.reshape(-1), gmat, xmat, umat, vmat,
             ag).reshape(epad, 8)


def kernel(x, weights, edge_index, params):
    n = x.shape[0]
    npad = ((n + N_PAD_BLK - 1) // N_PAD_BLK) * N_PAD_BLK
    xp = jnp.pad(x, ((0, npad - n), (0, 0)))
    src = edge_index[0]
    dst = edge_index[1]

    e = src.shape[0]
    epad = ((e + 8 * SC_NW * K1_C - 1) // (8 * SC_NW * K1_C)) * (8 * SC_NW * K1_C)
    srcp = jnp.concatenate([src, jnp.zeros((epad - e,), jnp.int32)])
    dstp = jnp.concatenate([dst, jnp.full((epad - e,), npad - 1, jnp.int32)])

    wbig = _weight_prep(params)
    sc8, G, U, V = _node_prep(xp, wbig)
    ag = params['gat_generalized_linear']['ag']
    lo8 = _k1_logits(srcp, dstp, sc8, G, xp, U, V, ag)

    ones = jnp.ones((e,), jnp.float32)
    deg = jax.ops.segment_sum(ones, dst, num_segments=npad)
    invdeg = (1.0 / jnp.clip(deg, 1.0))[:, None]
    dinv = 1.0 / jnp.sqrt(deg + 1.0)
    dinv2 = (dinv * dinv)[:, None]

    lo = lo8[:e, :6].T

    def seg_softmax(logits):
        m = jax.ops.segment_max(logits, dst, num_segments=npad)
        m = jnp.where(jnp.isfinite(m), m, 0.0)
        ee = jnp.exp(logits - m[dst])
        den = jax.ops.segment_sum(ee, dst, num_segments=npad)
        return ee / (den[dst] + 1e-16)

    att = jax.vmap(seg_softmax)(lo)
    norm = dinv[src] * dinv[dst]

    xs = xp[src]
    a1 = jax.ops.segment_sum(xs, dst, num_segments=npad)
    amax = jax.ops.segment_max(xs, dst, num_segments=npad)
    amax = jnp.where(jnp.isfinite(amax), amax, 0.0)
    agcn = jax.ops.segment_sum(xs * norm[:, None], dst, num_segments=npad)
    aatt = jax.vmap(
        lambda a: jax.ops.segment_sum(xs * a[:, None], dst,
                                      num_segments=npad))(att)

    res = _final(xp, a1, amax, agcn, aatt, invdeg, dinv2, weights, params)
    return res[:n]


# speedup vs baseline: 2.3461x; 2.2824x over previous
import functools

import jax
import jax.numpy as jnp
from jax import lax
from jax.experimental import pallas as pl
from jax.experimental.pallas import tpu as pltpu
from jax.experimental.pallas import tpu_sc as plsc

N_PAD_BLK = 1024

GATLIKE = ['gat', 'gat_sym', 'gat_linear', 'geniepath']


def _weight_prep_body(wcos_ref, alcos_ref, arcos_ref, wgen_ref, algen_ref,
                      argen_ref, wsc_ref, avec_ref, wbig_ref):
    wsc = wsc_ref[...]
    av = avec_ref[...]
    cols = []
    for p in range(4):
        cols.append(jnp.dot(wsc[p], av[p, 0][:, None],
                            preferred_element_type=jnp.float32))
        cols.append(jnp.dot(wsc[p], av[p, 1][:, None],
                            preferred_element_type=jnp.float32))
    sc_cols = jnp.concatenate(cols, axis=1)
    wcos = wcos_ref[...]
    mcos = jnp.dot(wcos * (alcos_ref[...] * arcos_ref[...])[None, :],
                   wcos.T, preferred_element_type=jnp.float32)
    wgen = wgen_ref[...]
    ucols = wgen * algen_ref[...][None, :]
    vcols = wgen * argen_ref[...][None, :]
    pad = jnp.zeros((256, 120), dtype=jnp.float32)
    wbig_ref[...] = jnp.concatenate([sc_cols, mcos, ucols, vcols, pad], axis=1)


def _weight_prep(p):
    wsc = jnp.stack([p[n]['W'] for n in GATLIKE])
    avec = jnp.stack([jnp.stack([p[n]['al'], p[n]['ar']]) for n in GATLIKE])
    return pl.pallas_call(
        _weight_prep_body,
        out_shape=jax.ShapeDtypeStruct((256, 896), jnp.float32),
    )(p['gat_cos']['W'], p['gat_cos']['al'], p['gat_cos']['ar'],
      p['gat_generalized_linear']['W'], p['gat_generalized_linear']['al'],
      p['gat_generalized_linear']['ar'], wsc, avec)


def _node_prep_body(x_ref, wbig_ref, sc_ref, g_ref, u_ref, v_ref):
    h = jnp.dot(x_ref[...], wbig_ref[...], preferred_element_type=jnp.float32)
    sc_ref[...] = h[:, 0:8]
    g_ref[...] = h[:, 8:264]
    u_ref[...] = h[:, 264:520]
    v_ref[...] = h[:, 520:776]


def _node_prep(xp, wbig):
    npad = xp.shape[0]
    blk = lambda w: pl.BlockSpec((N_PAD_BLK, w), lambda i: (i, 0))
    return pl.pallas_call(
        _node_prep_body,
        grid=(npad // N_PAD_BLK,),
        in_specs=[pl.BlockSpec((N_PAD_BLK, 256), lambda i: (i, 0)),
                  pl.BlockSpec((256, 896), lambda i: (0, 0))],
        out_specs=[blk(8), blk(256), blk(256), blk(256)],
        out_shape=[jax.ShapeDtypeStruct((npad, 8), jnp.float32),
                   jax.ShapeDtypeStruct((npad, 256), jnp.float32),
                   jax.ShapeDtypeStruct((npad, 256), jnp.float32),
                   jax.ShapeDtypeStruct((npad, 256), jnp.float32)],
    )(xp, wbig)


def _final_body(x_ref, a1_ref, amax_ref, agcn_ref, aatt_ref, invdeg_ref,
                dinv2_ref, wmix_ref, wl_ref, wr_ref, bs_ref, wgcn_ref,
                bgcn_ref, w1_ref, b1_ref, w2_ref, b2_ref, eps_ref, wgat_ref,
                bgat_ref, wgen_ref, wlstm_ref, blstm_ref, out_ref):
    x = x_ref[...]
    a1 = a1_ref[...]
    invdeg = invdeg_ref[...]
    wmix = wmix_ref[...]

    def elu(v):
        return jnp.where(v > 0, v, jnp.exp(jnp.minimum(v, 0.0)) - 1.0)

    def mm(a, w):
        return jnp.dot(a, w, preferred_element_type=jnp.float32)

    acc = jnp.zeros_like(x)
    aggs = [a1 * invdeg, a1, amax_ref[...]]
    for k in range(3):
        out = mm(aggs[k], wl_ref[k]) + mm(x, wr_ref[k]) + bs_ref[k][None, :]
        acc += wmix[0, k] * elu(out)
    out = mm(agcn_ref[...] + dinv2_ref[...] * x, wgcn_ref[...]) \
        + bgcn_ref[...][None, :]
    acc += wmix[0, 3] * elu(out)
    h = (1.0 + eps_ref[0, 0]) * x + a1
    h = jnp.maximum(mm(h, w1_ref[...]) + b1_ref[...][None, :], 0.0)
    out = mm(h, w2_ref[...]) + b2_ref[...][None, :]
    acc += wmix[0, 4] * elu(out)
    for k in range(5):
        out = mm(aatt_ref[k], wgat_ref[k]) + bgat_ref[k][None, :]
        acc += wmix[0, 5 + k] * elu(out)
    hb = jnp.tanh(mm(aatt_ref[5], wgen_ref[...]))
    gates = mm(hb, wlstm_ref[...]) + blstm_ref[...][None, :]
    i = gates[:, 0:256]
    g = gates[:, 512:768]
    o = gates[:, 768:1024]
    c = jax.nn.sigmoid(i) * jnp.tanh(g)
    out = jax.nn.sigmoid(o) * jnp.tanh(c)
    acc += wmix[0, 10] * elu(out)
    out_ref[...] = acc


def _final(xp, a1, amax, agcn, aatt, invdeg, dinv2, weights, p):
    npad = xp.shape[0]
    nb = npad // N_PAD_BLK
    wl = jnp.stack([p[n]['Wl'] for n in ['sage', 'sage_sum', 'sage_max']])
    wr = jnp.stack([p[n]['Wr'] for n in ['sage', 'sage_sum', 'sage_max']])
    bs = jnp.stack([p[n]['b'] for n in ['sage', 'sage_sum', 'sage_max']])
    gat_names = ['gat', 'gat_sym', 'gat_cos', 'gat_linear',
                 'gat_generalized_linear']
    wgat = jnp.stack([p[n]['W'] for n in gat_names])
    bgat = jnp.stack([p[n]['b'] for n in gat_names])
    wmix = weights.reshape(1, 11)
    eps = p['gin']['eps'].reshape(1, 1)

    full = lambda *s: pl.BlockSpec(s, lambda i: (0,) * len(s))
    blk = pl.BlockSpec((N_PAD_BLK, 256), lambda i: (i, 0))
    blk1 = pl.BlockSpec((N_PAD_BLK, 1), lambda i: (i, 0))
    in_specs = [
        blk, blk,
        blk, blk,
        pl.BlockSpec((6, N_PAD_BLK, 256), lambda i: (0, i, 0)),
        blk1, blk1,
        full(1, 11),
        full(3, 256, 256), full(3, 256, 256), full(3, 256),
        full(256, 256), full(256,),
        full(256, 256), full(256,), full(256, 256), full(256,), full(1, 1),
        full(5, 256, 256), full(5, 256),
        full(256, 256), full(256, 1024), full(1024,),
    ]
    return pl.pallas_call(
        _final_body,
        grid=(nb,),
        in_specs=in_specs,
        out_specs=blk,
        out_shape=jax.ShapeDtypeStruct((npad, 256), jnp.float32),
    )(xp, a1, amax, agcn, aatt, invdeg, dinv2, wmix, wl, wr, bs,
      p['gcn']['W'], p['gcn']['b'], p['gin']['W1'], p['gin']['b1'],
      p['gin']['W2'], p['gin']['b2'], eps, wgat, bgat,
      p['geniepath']['W'], p['geniepath']['Wlstm'], p['geniepath']['blstm'])


SC_NC = 2
SC_NS = 16
SC_NW = SC_NC * SC_NS
K1_C = 32


def _k1_body(srcp, dstp, sc8f, gmat, xmat, umat, vmat, agvec, lo_out,
             src_v, dst_v, tab_v, g_v, x_v, u_v, v_v, ag_v, lo_v,
             part_v, sem):
    epw = srcp.shape[0] // SC_NW
    npad8 = sc8f.shape[0]
    wid = lax.axis_index("s") * SC_NC + lax.axis_index("c")
    base_w = wid * epw
    pltpu.sync_copy(agvec, ag_v)
    pltpu.sync_copy(sc8f, tab_v)

    def zinit(i, c):
        lo_v[pl.ds(i * 16, 16)] = jnp.zeros((16,), jnp.float32)
        return c

    lax.fori_loop(0, K1_C, zinit, 0, unroll=False)

    def lrelu(z):
        return jnp.where(z > 0, z, 0.2 * z)

    def tanh16(z):
        e2 = jnp.exp(z + z)
        return 1.0 - 2.0 / (e2 + 1.0)

    def chunk(i, carry):
        base = base_w + i * K1_C
        pltpu.sync_copy(srcp.at[pl.ds(base, K1_C)], src_v)
        pltpu.sync_copy(dstp.at[pl.ds(base, K1_C)], dst_v)
        pltpu.async_copy(gmat.at[src_v], g_v, sem).wait()
        pltpu.async_copy(xmat.at[dst_v], x_v, sem).wait()
        pltpu.async_copy(umat.at[src_v], u_v, sem).wait()
        pltpu.async_copy(vmat.at[dst_v], v_v, sem).wait()

        def edge(e, carry2):
            acc_c = jnp.zeros((16,), jnp.float32)
            acc_g = jnp.zeros((16,), jnp.float32)
            for k in range(16):
                ks = pl.ds(k * 16, 16)
                acc_c += g_v[e, ks] * x_v[e, ks]
                z = u_v[e, ks] + v_v[e, ks]
                acc_g += tanh16(z) * ag_v[ks]
            part_v[pl.ds(e * 16, 16)] = acc_c
            part_v[pl.ds(K1_C * 16 + e * 16, 16)] = acc_g
            return carry2

        lax.fori_loop(0, K1_C, edge, 0, unroll=False)

        for g16 in range(K1_C // 16):
            rows = lax.iota(jnp.int32, 16) + g16 * 16
            s16 = src_v[pl.ds(g16 * 16, 16)]
            d16 = dst_v[pl.ds(g16 * 16, 16)]

            def sf(n16, f):
                return plsc.load_gather(tab_v, [n16 * 8 + f])

            lo_gat = lrelu(sf(s16, 0) + sf(d16, 1))
            lo_sym = (lrelu(sf(s16, 2) + sf(d16, 3))
                      + lrelu(sf(d16, 2) + sf(s16, 3)))
            lo_lin = tanh16(sf(s16, 4) + sf(d16, 5))
            lo_gen8 = lrelu(sf(s16, 6) + sf(d16, 7))
            lo_cos = jnp.zeros((16,), jnp.float32)
            lo_gnl = jnp.zeros((16,), jnp.float32)
            for k in range(16):
                lo_cos += plsc.load_gather(part_v, [rows * 16 + k])
                lo_gnl += plsc.load_gather(part_v, [K1_C * 16 + rows * 16 + k])
            for f, val in ((0, lo_gat), (1, lo_sym), (2, lo_cos),
                           (3, lo_lin), (4, lo_gnl), (5, lo_gen8)):
                plsc.store_scatter(lo_v, [rows * 16 + f], val)
        pltpu.sync_copy(lo_v, lo_out.at[pl.ds(base * 16, K1_C * 16)])
        return carry

    lax.fori_loop(0, epw // K1_C, chunk, 0, unroll=False)


def _k1_logits(srcp, dstp, sc8, gmat, xmat, umat, vmat, ag):
    epad = srcp.shape[0]
    npad = sc8.shape[0]
    mesh = plsc.VectorSubcoreMesh(core_axis_name="c", subcore_axis_name="s")
    f = pl.kernel(
        _k1_body,
        out_type=jax.ShapeDtypeStruct((epad * 16,), jnp.float32),
        mesh=mesh,
        compiler_params=pltpu.CompilerParams(needs_layout_passes=False),
        scratch_types=[
            pltpu.VMEM((K1_C,), jnp.int32),
            pltpu.VMEM((K1_C,), jnp.int32),
            pltpu.VMEM((npad * 8,), jnp.float32),
            pltpu.VMEM((K1_C, 256), jnp.float32),
            pltpu.VMEM((K1_C, 256), jnp.float32),
            pltpu.VMEM((K1_C, 256), jnp.float32),
            pltpu.VMEM((K1_C, 256), jnp.float32),
            pltpu.VMEM((256,), jnp.float32),
            pltpu.VMEM((K1_C * 16,), jnp.float32),
            pltpu.VMEM((2 * K1_C * 16,), jnp.float32),
            pltpu.SemaphoreType.DMA,
        ],
    )
    return f(srcp, dstp, sc8.reshape(-1), gmat, xmat, umat, vmat, ag)


K2_C = 128


def _k2_body(dstp, lo16f, zvec, maxpart, degpart,
             dst_v, lo_v, bins_v, ones_v, degacc, sem):
    epw = dstp.shape[0] // SC_NW
    npad = zvec.shape[0]
    cid = lax.axis_index("c")
    sid = lax.axis_index("s")
    wid = sid * SC_NC + cid
    base_w = wid * epw
    neg = jnp.full((16,), -jnp.inf, jnp.float32)

    def binit(i, c):
        bins_v[pl.ds(i * 16, 16)] = neg
        return c

    lax.fori_loop(0, npad * 8 // 16, binit, 0, unroll=False)

    def oinit(i, c):
        ones_v[pl.ds(i * 16, 16)] = jnp.ones((16,), jnp.float32)
        return c

    lax.fori_loop(0, K2_C // 16, oinit, 0, unroll=False)

    @pl.when(sid == 0)
    def _():
        pltpu.sync_copy(zvec, degacc)

    plsc.subcore_barrier()
    q16 = lax.iota(jnp.int32, 16)
    eoff = q16 >> 3
    fld = q16 & 7

    def chunk(i, carry):
        base = base_w + i * K2_C
        pltpu.sync_copy(dstp.at[pl.ds(base, K2_C)], dst_v)
        pltpu.sync_copy(lo16f.at[pl.ds(base * 16, K2_C * 16)], lo_v)
        pltpu.sync_copy(ones_v, degacc.at[dst_v], add=True)

        def pair(p, c2):
            e_ids = p * 2 + eoff
            esw = e_ids ^ 1
            d16 = plsc.load_gather(dst_v, [e_ids])
            dsw = plsc.load_gather(dst_v, [esw])
            lov = plsc.load_gather(lo_v, [e_ids * 16 + fld])
            losw = plsc.load_gather(lo_v, [esw * 16 + fld])
            val = jnp.where(d16 == dsw, jnp.maximum(lov, losw), lov)
            idx = d16 * 8 + fld
            cur = plsc.load_gather(bins_v, [idx])
            plsc.store_scatter(bins_v, [idx], jnp.maximum(cur, val))
            return c2

        lax.fori_loop(0, K2_C // 2, pair, 0, unroll=False)
        return carry

    lax.fori_loop(0, epw // K2_C, chunk, 0, unroll=False)
    pltpu.sync_copy(bins_v, maxpart.at[wid])
    plsc.subcore_barrier()
    nseg = npad // SC_NS
    pltpu.sync_copy(degacc.at[pl.ds(sid * nseg, nseg)],
                    degpart.at[cid, pl.ds(sid * nseg, nseg)])


def _k2_maxdeg(dstp, lo16f, npad):
    epad = dstp.shape[0]
    mesh = plsc.VectorSubcoreMesh(core_axis_name="c", subcore_axis_name="s")
    f = pl.kernel(
        _k2_body,
        out_type=[jax.ShapeDtypeStruct((SC_NW, npad * 8), jnp.float32),
                  jax.ShapeDtypeStruct((SC_NC, npad), jnp.float32)],
        mesh=mesh,
        compiler_params=pltpu.CompilerParams(needs_layout_passes=False),
        scratch_types=[
            pltpu.VMEM((K2_C,), jnp.int32),
            pltpu.VMEM((K2_C * 16,), jnp.float32),
            pltpu.VMEM((npad * 8,), jnp.float32),
            pltpu.VMEM((K2_C,), jnp.float32),
            pltpu.VMEM_SHARED((npad,), jnp.float32),
            pltpu.SemaphoreType.DMA,
        ],
    )
    return f(dstp, lo16f, jnp.zeros((npad,), jnp.float32))


K3_C = 128


def _k3a_body(srcp, dstp, lo16f, mtab8f, numf_out,
              src_v, dst_v, lo_v, numf_v, mtab_v, sem):
    epw = srcp.shape[0] // SC_NW
    wid = lax.axis_index("s") * SC_NC + lax.axis_index("c")
    base_w = wid * epw
    pltpu.sync_copy(mtab8f, mtab_v)
    q16 = lax.iota(jnp.int32, 16)
    f8 = q16 & 7

    def chunk(i, carry):
        base = base_w + i * K3_C
        pltpu.sync_copy(srcp.at[pl.ds(base, K3_C)], src_v)
        pltpu.sync_copy(dstp.at[pl.ds(base, K3_C)], dst_v)
        pltpu.sync_copy(lo16f.at[pl.ds(base * 16, K3_C * 16)], lo_v)

        def edge(e, c2):
            e16 = jnp.full((16,), e, jnp.int32)
            d16 = plsc.load_gather(dst_v, [e16])
            s16 = plsc.load_gather(src_v, [e16])
            m16 = plsc.load_gather(mtab_v, [d16 * 8 + f8])
            dinv_d = plsc.load_gather(mtab_v, [d16 * 8 + 6])
            dinv_s = plsc.load_gather(mtab_v, [s16 * 8 + 6])
            lo16 = lo_v[pl.ds(e * 16, 16)]
            num16 = jnp.exp(lo16 - m16)
            num16 = jnp.where(q16 == 6, dinv_s * dinv_d, num16)
            num16 = jnp.where(q16 < 8, num16, 0.0)
            numf_v[pl.ds(e * 16, 16)] = num16
            return c2

        lax.fori_loop(0, K3_C, edge, 0, unroll=False)
        pltpu.sync_copy(numf_v, numf_out.at[pl.ds(base * 16, K3_C * 16)])
        return carry

    lax.fori_loop(0, epw // K3_C, chunk, 0, unroll=False)


def _k3b_body(dstp, numf, denpart, dst_v, numf_v, bins_v, sem):
    epw = dstp.shape[0] // SC_NW
    npad = denpart.shape[1] // 8
    wid = lax.axis_index("s") * SC_NC + lax.axis_index("c")
    base_w = wid * epw

    def binit(i, c):
        bins_v[pl.ds(i * 16, 16)] = jnp.zeros((16,), jnp.float32)
        return c

    lax.fori_loop(0, npad * 8 // 16, binit, 0, unroll=False)
    q16 = lax.iota(jnp.int32, 16)
    eoff = q16 >> 3
    fld = q16 & 7

    def chunk(i, carry):
        base = base_w + i * K3_C
        pltpu.sync_copy(dstp.at[pl.ds(base, K3_C)], dst_v)
        pltpu.sync_copy(numf.at[pl.ds(base * 16, K3_C * 16)], numf_v)

        def pair(p, c2):
            e_ids = p * 2 + eoff
            esw = e_ids ^ 1
            d16 = plsc.load_gather(dst_v, [e_ids])
            dsw = plsc.load_gather(dst_v, [esw])
            nv = plsc.load_gather(numf_v, [e_ids * 16 + fld])
            nsw = plsc.load_gather(numf_v, [esw * 16 + fld])
            val = jnp.where(d16 == dsw, nv + nsw, nv)
            idx = d16 * 8 + fld
            cur = plsc.load_gather(bins_v, [idx])
            plsc.store_scatter(bins_v, [idx], cur + val)
            return c2

        lax.fori_loop(0, K3_C // 2, pair, 0, unroll=False)
        return carry

    lax.fori_loop(0, epw // K3_C, chunk, 0, unroll=False)
    pltpu.sync_copy(bins_v, denpart.at[wid])


def _k3_num(srcp, dstp, lo16f, mtab8, npad):
    epad = srcp.shape[0]
    mesh = plsc.VectorSubcoreMesh(core_axis_name="c", subcore_axis_name="s")
    fa = pl.kernel(
        _k3a_body,
        out_type=jax.ShapeDtypeStruct((epad * 16,), jnp.float32),
        mesh=mesh,
        compiler_params=pltpu.CompilerParams(needs_layout_passes=False),
        scratch_types=[
            pltpu.VMEM((K3_C,), jnp.int32),
            pltpu.VMEM((K3_C,), jnp.int32),
            pltpu.VMEM((K3_C * 16,), jnp.float32),
            pltpu.VMEM((K3_C * 16,), jnp.float32),
            pltpu.VMEM((npad * 8,), jnp.float32),
            pltpu.SemaphoreType.DMA,
        ],
    )
    numf = fa(srcp, dstp, lo16f, mtab8.reshape(-1))
    fb = pl.kernel(
        _k3b_body,
        out_type=jax.ShapeDtypeStruct((SC_NW, npad * 8), jnp.float32),
        mesh=mesh,
        compiler_params=pltpu.CompilerParams(needs_layout_passes=False),
        scratch_types=[
            pltpu.VMEM((K3_C,), jnp.int32),
            pltpu.VMEM((K3_C * 16,), jnp.float32),
            pltpu.VMEM((npad * 8,), jnp.float32),
            pltpu.SemaphoreType.DMA,
        ],
    )
    denpart = fb(dstp, numf)
    return numf, denpart


def _mid2_body(denpart_ref, stab_ref):
    den = jnp.sum(denpart_ref[...], axis=0)
    inv = 1.0 / (den + 1e-16)
    q = lax.broadcasted_iota(jnp.int32, inv.shape, 1)
    stab_ref[...] = jnp.where(q < 6, inv, 1.0)


def _mid2(denpart, npad):
    nb = npad // N_PAD_BLK
    denp = denpart.reshape(SC_NW, npad, 8)
    return pl.pallas_call(
        _mid2_body,
        grid=(nb,),
        in_specs=[pl.BlockSpec((SC_NW, N_PAD_BLK, 8), lambda i: (0, i, 0))],
        out_specs=pl.BlockSpec((N_PAD_BLK, 8), lambda i: (i, 0)),
        out_shape=jax.ShapeDtypeStruct((npad, 8), jnp.float32),
    )(denp)


def _mid_body(maxpart_ref, degpart_ref, mtab_ref, invdeg_ref, dinv2_ref):
    m = jnp.max(maxpart_ref[...], axis=0)
    m = jnp.where(jnp.isfinite(m), m, 0.0)
    deg = degpart_ref[0] + degpart_ref[1]
    dinv = 1.0 / jnp.sqrt(deg + 1.0)
    blk = m.shape[0]
    mtab_ref[...] = jnp.concatenate(
        [m[:, :6], dinv[:, None], jnp.zeros((blk, 1), jnp.float32)], axis=1)
    invdeg_ref[...] = (1.0 / jnp.clip(deg, 1.0))[:, None]
    dinv2_ref[...] = (dinv * dinv)[:, None]


def _mid(maxpart, degpart, npad):
    nb = npad // N_PAD_BLK
    maxp = maxpart.reshape(SC_NW, npad, 8)
    return pl.pallas_call(
        _mid_body,
        grid=(nb,),
        in_specs=[pl.BlockSpec((SC_NW, N_PAD_BLK, 8), lambda i: (0, i, 0)),
                  pl.BlockSpec((SC_NC, N_PAD_BLK), lambda i: (0, i))],
        out_specs=[pl.BlockSpec((N_PAD_BLK, 8), lambda i: (i, 0)),
                   pl.BlockSpec((N_PAD_BLK, 1), lambda i: (i, 0)),
                   pl.BlockSpec((N_PAD_BLK, 1), lambda i: (i, 0))],
        out_shape=[jax.ShapeDtypeStruct((npad, 8), jnp.float32),
                   jax.ShapeDtypeStruct((npad, 1), jnp.float32),
                   jax.ShapeDtypeStruct((npad, 1), jnp.float32)],
    )(maxp, degpart)


def _k3c_body(dstp, numf, stab8f, wf_out, dst_v, numf_v, w_v, stab_v, sem):
    epw = dstp.shape[0] // SC_NW
    wid = lax.axis_index("s") * SC_NC + lax.axis_index("c")
    base_w = wid * epw
    pltpu.sync_copy(stab8f, stab_v)

    def zinit(i, c):
        w_v[pl.ds(i * 16, 16)] = jnp.zeros((16,), jnp.float32)
        return c

    lax.fori_loop(0, K3_C, zinit, 0, unroll=False)
    q16 = lax.iota(jnp.int32, 16)
    eoff = q16 >> 3
    fld = q16 & 7

    def chunk(i, carry):
        base = base_w + i * K3_C
        pltpu.sync_copy(dstp.at[pl.ds(base, K3_C)], dst_v)
        pltpu.sync_copy(numf.at[pl.ds(base * 16, K3_C * 16)], numf_v)

        def pair(p, c2):
            e_ids = p * 2 + eoff
            d16 = plsc.load_gather(dst_v, [e_ids])
            nv = plsc.load_gather(numf_v, [e_ids * 16 + fld])
            sv = plsc.load_gather(stab_v, [d16 * 8 + fld])
            plsc.store_scatter(w_v, [e_ids * 16 + fld], nv * sv)
            return c2

        lax.fori_loop(0, K3_C // 2, pair, 0, unroll=False)
        pltpu.sync_copy(w_v, wf_out.at[pl.ds(base * 16, K3_C * 16)])
        return carry

    lax.fori_loop(0, epw // K3_C, chunk, 0, unroll=False)


def _k3c_w(dstp, numf, stab8, npad):
    epad = dstp.shape[0]
    mesh = plsc.VectorSubcoreMesh(core_axis_name="c", subcore_axis_name="s")
    f = pl.kernel(
        _k3c_body,
        out_type=jax.ShapeDtypeStruct((epad * 16,), jnp.float32),
        mesh=mesh,
        compiler_params=pltpu.CompilerParams(needs_layout_passes=False),
        scratch_types=[
            pltpu.VMEM((K3_C,), jnp.int32),
            pltpu.VMEM((K3_C * 16,), jnp.float32),
            pltpu.VMEM((K3_C * 16,), jnp.float32),
            pltpu.VMEM((npad * 8,), jnp.float32),
            pltpu.SemaphoreType.DMA,
        ],
    )
    return f(dstp, numf, stab8.reshape(-1))


K5_C = 128


def _k5_body(srcp, dstp, wf, xh2, zmat, apart,
             src_v, idx_v, dst_v, w_v, x_v, out_v, acc, sem):
    epad = srcp.shape[0]
    epw = epad // SC_NS
    npad = zmat.shape[0]
    cid = lax.axis_index("c")
    sid = lax.axis_index("s")
    base_w = sid * epw
    off = cid * npad
    nseg = npad // SC_NS

    for p in range(8):
        @pl.when(sid == 0)
        def _():
            pltpu.sync_copy(zmat, acc)

        plsc.subcore_barrier()

        def chunk(i, carry):
            base = base_w + i * K5_C
            pltpu.sync_copy(srcp.at[pl.ds(base, K5_C)], src_v)
            pltpu.sync_copy(dstp.at[pl.ds(base, K5_C)], dst_v)
            pltpu.sync_copy(wf.at[pl.ds(base * 16, K5_C * 16)], w_v)

            def adj(j, c2):
                idx_v[pl.ds(j * 16, 16)] = src_v[pl.ds(j * 16, 16)] + off
                return c2

            lax.fori_loop(0, K5_C // 16, adj, 0, unroll=False)
            pltpu.async_copy(xh2.at[idx_v], x_v, sem).wait()

            def edge(e, c2):
                wsplat = plsc.load_gather(
                    w_v, [jnp.full((16,), e * 16 + p, jnp.int32)])
                for t in range(8):
                    ts = pl.ds(t * 16, 16)
                    out_v[e, ts] = wsplat * x_v[e, ts]
                return c2

            lax.fori_loop(0, K5_C, edge, 0, unroll=False)
            pltpu.sync_copy(out_v, acc.at[dst_v], add=True)
            return carry

        lax.fori_loop(0, epw // K5_C, chunk, 0, unroll=False)
        plsc.subcore_barrier()
        pltpu.sync_copy(acc.at[pl.ds(sid * nseg, nseg)],
                        apart.at[cid, p, pl.ds(sid * nseg, nseg)])
        plsc.subcore_barrier()


def _k5_accs(srcp, dstp, wf, xp, npad):
    epad = srcp.shape[0]
    xh2 = jnp.concatenate([xp[:, :128], xp[:, 128:]], axis=0)
    mesh = plsc.VectorSubcoreMesh(core_axis_name="c", subcore_axis_name="s")
    f = pl.kernel(
        _k5_body,
        out_type=jax.ShapeDtypeStruct((SC_NC, 8, npad, 128), jnp.float32),
        mesh=mesh,
        compiler_params=pltpu.CompilerParams(needs_layout_passes=False),
        scratch_types=[
            pltpu.VMEM((K5_C,), jnp.int32),
            pltpu.VMEM((K5_C,), jnp.int32),
            pltpu.VMEM((K5_C,), jnp.int32),
            pltpu.VMEM((K5_C * 16,), jnp.float32),
            pltpu.VMEM((K5_C, 128), jnp.float32),
            pltpu.VMEM((K5_C, 128), jnp.float32),
            pltpu.VMEM_SHARED((npad, 128), jnp.float32),
            pltpu.SemaphoreType.DMA,
        ],
    )
    apart = f(srcp, dstp, wf, xh2, jnp.zeros((npad, 128), jnp.float32))
    return jnp.concatenate([apart[0], apart[1]], axis=-1)


def kernel(x, weights, edge_index, params):
    n = x.shape[0]
    npad = ((n + N_PAD_BLK - 1) // N_PAD_BLK) * N_PAD_BLK
    xp = jnp.pad(x, ((0, npad - n), (0, 0)))
    src = edge_index[0]
    dst = edge_index[1]

    e = src.shape[0]
    epad = ((e + 8 * SC_NW * K1_C - 1) // (8 * SC_NW * K1_C)) * (8 * SC_NW * K1_C)
    srcp = jnp.concatenate([src, jnp.zeros((epad - e,), jnp.int32)])
    dstp = jnp.concatenate([dst, jnp.full((epad - e,), npad - 1, jnp.int32)])

    wbig = _weight_prep(params)
    sc8, G, U, V = _node_prep(xp, wbig)
    ag = params['gat_generalized_linear']['ag']
    lo16f = _k1_logits(srcp, dstp, sc8, G, xp, U, V, ag)

    maxpart, degpart = _k2_maxdeg(dstp, lo16f, npad)
    mtab, invdeg, dinv2 = _mid(maxpart, degpart, npad)

    numf, denpart = _k3_num(srcp, dstp, lo16f, mtab, npad)
    stab = _mid2(denpart, npad)
    wf = _k3c_w(dstp, numf, stab, npad)
    a8 = _k5_accs(srcp, dstp, wf, xp, npad)
    aatt = a8[:6]
    agcn = a8[6]
    a1 = a8[7]

    xs = xp[src]
    amax = jax.ops.segment_max(xs, dst, num_segments=npad)
    amax = jnp.where(jnp.isfinite(amax), amax, 0.0)

    res = _final(xp, a1, amax, agcn, aatt, invdeg, dinv2, weights, params)
    return res[:n]


# speedup vs baseline: 2.9925x; 1.2755x over previous
import functools

import jax
import jax.numpy as jnp
from jax import lax
from jax.experimental import pallas as pl
from jax.experimental.pallas import tpu as pltpu
from jax.experimental.pallas import tpu_sc as plsc

N_PAD_BLK = 1024

GATLIKE = ['gat', 'gat_sym', 'gat_linear', 'geniepath']


def _weight_prep_body(wcos_ref, alcos_ref, arcos_ref, wgen_ref, algen_ref,
                      argen_ref, wsc_ref, avec_ref, wbig_ref):
    wsc = wsc_ref[...]
    av = avec_ref[...]
    cols = []
    for p in range(4):
        cols.append(jnp.dot(wsc[p], av[p, 0][:, None],
                            preferred_element_type=jnp.float32))
        cols.append(jnp.dot(wsc[p], av[p, 1][:, None],
                            preferred_element_type=jnp.float32))
    sc_cols = jnp.concatenate(cols, axis=1)
    wcos = wcos_ref[...]
    mcos = jnp.dot(wcos * (alcos_ref[...] * arcos_ref[...])[None, :],
                   wcos.T, preferred_element_type=jnp.float32)
    wgen = wgen_ref[...]
    ucols = wgen * algen_ref[...][None, :]
    vcols = wgen * argen_ref[...][None, :]
    pad = jnp.zeros((256, 120), dtype=jnp.float32)
    wbig_ref[...] = jnp.concatenate([sc_cols, mcos, ucols, vcols, pad], axis=1)


def _weight_prep(p):
    wsc = jnp.stack([p[n]['W'] for n in GATLIKE])
    avec = jnp.stack([jnp.stack([p[n]['al'], p[n]['ar']]) for n in GATLIKE])
    return pl.pallas_call(
        _weight_prep_body,
        out_shape=jax.ShapeDtypeStruct((256, 896), jnp.float32),
    )(p['gat_cos']['W'], p['gat_cos']['al'], p['gat_cos']['ar'],
      p['gat_generalized_linear']['W'], p['gat_generalized_linear']['al'],
      p['gat_generalized_linear']['ar'], wsc, avec)


def _node_prep_body(x_ref, wbig_ref, sc_ref, g_ref, u_ref, v_ref):
    h = jnp.dot(x_ref[...], wbig_ref[...], preferred_element_type=jnp.float32)
    sc_ref[...] = h[:, 0:8]
    g_ref[...] = h[:, 8:264]
    u_ref[...] = h[:, 264:520]
    v_ref[...] = h[:, 520:776]


def _node_prep(xp, wbig):
    npad = xp.shape[0]
    blk = lambda w: pl.BlockSpec((N_PAD_BLK, w), lambda i: (i, 0))
    return pl.pallas_call(
        _node_prep_body,
        grid=(npad // N_PAD_BLK,),
        in_specs=[pl.BlockSpec((N_PAD_BLK, 256), lambda i: (i, 0)),
                  pl.BlockSpec((256, 896), lambda i: (0, 0))],
        out_specs=[blk(8), blk(256), blk(256), blk(256)],
        out_shape=[jax.ShapeDtypeStruct((npad, 8), jnp.float32),
                   jax.ShapeDtypeStruct((npad, 256), jnp.float32),
                   jax.ShapeDtypeStruct((npad, 256), jnp.float32),
                   jax.ShapeDtypeStruct((npad, 256), jnp.float32)],
    )(xp, wbig)


def _final_body(x_ref, a1_ref, amax_ref, agcn_ref, aatt_ref, invdeg_ref,
                dinv2_ref, wmix_ref, wl_ref, wr_ref, bs_ref, wgcn_ref,
                bgcn_ref, w1_ref, b1_ref, w2_ref, b2_ref, eps_ref, wgat_ref,
                bgat_ref, wgen_ref, wlstm_ref, blstm_ref, out_ref):
    x = x_ref[...]
    a1 = a1_ref[...]
    invdeg = invdeg_ref[...]
    wmix = wmix_ref[...]

    def elu(v):
        return jnp.where(v > 0, v, jnp.exp(jnp.minimum(v, 0.0)) - 1.0)

    def mm(a, w):
        return jnp.dot(a, w, preferred_element_type=jnp.float32)

    acc = jnp.zeros_like(x)
    aggs = [a1 * invdeg, a1, amax_ref[...]]
    for k in range(3):
        out = mm(aggs[k], wl_ref[k]) + mm(x, wr_ref[k]) + bs_ref[k][None, :]
        acc += wmix[0, k] * elu(out)
    out = mm(agcn_ref[...] + dinv2_ref[...] * x, wgcn_ref[...]) \
        + bgcn_ref[...][None, :]
    acc += wmix[0, 3] * elu(out)
    h = (1.0 + eps_ref[0, 0]) * x + a1
    h = jnp.maximum(mm(h, w1_ref[...]) + b1_ref[...][None, :], 0.0)
    out = mm(h, w2_ref[...]) + b2_ref[...][None, :]
    acc += wmix[0, 4] * elu(out)
    for k in range(5):
        out = mm(aatt_ref[k], wgat_ref[k]) + bgat_ref[k][None, :]
        acc += wmix[0, 5 + k] * elu(out)
    hb = jnp.tanh(mm(aatt_ref[5], wgen_ref[...]))
    gates = mm(hb, wlstm_ref[...]) + blstm_ref[...][None, :]
    i = gates[:, 0:256]
    g = gates[:, 512:768]
    o = gates[:, 768:1024]
    c = jax.nn.sigmoid(i) * jnp.tanh(g)
    out = jax.nn.sigmoid(o) * jnp.tanh(c)
    acc += wmix[0, 10] * elu(out)
    out_ref[...] = acc


def _final(xp, a1, amax, agcn, aatt, invdeg, dinv2, weights, p):
    npad = xp.shape[0]
    nb = npad // N_PAD_BLK
    wl = jnp.stack([p[n]['Wl'] for n in ['sage', 'sage_sum', 'sage_max']])
    wr = jnp.stack([p[n]['Wr'] for n in ['sage', 'sage_sum', 'sage_max']])
    bs = jnp.stack([p[n]['b'] for n in ['sage', 'sage_sum', 'sage_max']])
    gat_names = ['gat', 'gat_sym', 'gat_cos', 'gat_linear',
                 'gat_generalized_linear']
    wgat = jnp.stack([p[n]['W'] for n in gat_names])
    bgat = jnp.stack([p[n]['b'] for n in gat_names])
    wmix = weights.reshape(1, 11)
    eps = p['gin']['eps'].reshape(1, 1)

    full = lambda *s: pl.BlockSpec(s, lambda i: (0,) * len(s))
    blk = pl.BlockSpec((N_PAD_BLK, 256), lambda i: (i, 0))
    blk1 = pl.BlockSpec((N_PAD_BLK, 1), lambda i: (i, 0))
    in_specs = [
        blk, blk,
        blk, blk,
        pl.BlockSpec((6, N_PAD_BLK, 256), lambda i: (0, i, 0)),
        blk1, blk1,
        full(1, 11),
        full(3, 256, 256), full(3, 256, 256), full(3, 256),
        full(256, 256), full(256,),
        full(256, 256), full(256,), full(256, 256), full(256,), full(1, 1),
        full(5, 256, 256), full(5, 256),
        full(256, 256), full(256, 1024), full(1024,),
    ]
    return pl.pallas_call(
        _final_body,
        grid=(nb,),
        in_specs=in_specs,
        out_specs=blk,
        out_shape=jax.ShapeDtypeStruct((npad, 256), jnp.float32),
    )(xp, a1, amax, agcn, aatt, invdeg, dinv2, wmix, wl, wr, bs,
      p['gcn']['W'], p['gcn']['b'], p['gin']['W1'], p['gin']['b1'],
      p['gin']['W2'], p['gin']['b2'], eps, wgat, bgat,
      p['geniepath']['W'], p['geniepath']['Wlstm'], p['geniepath']['blstm'])


SC_NC = 2
SC_NS = 16
SC_NW = SC_NC * SC_NS
K1_C = 32


def _k1_body(srcp, dstp, sc8f, gmat, xmat, umat, vmat, agvec, lo_out,
             src_v, dst_v, tab_v, g_v, x_v, u_v, v_v, ag_v, lo_v,
             part_v, sem):
    epw = srcp.shape[0] // SC_NW
    npad8 = sc8f.shape[0]
    wid = lax.axis_index("s") * SC_NC + lax.axis_index("c")
    base_w = wid * epw
    pltpu.sync_copy(agvec, ag_v)
    pltpu.sync_copy(sc8f, tab_v)

    def zinit(i, c):
        lo_v[pl.ds(i * 16, 16)] = jnp.zeros((16,), jnp.float32)
        return c

    lax.fori_loop(0, K1_C, zinit, 0, unroll=False)

    def lrelu(z):
        return jnp.where(z > 0, z, 0.2 * z)

    def tanh16(z):
        e2 = jnp.exp(z + z)
        return 1.0 - 2.0 / (e2 + 1.0)

    def chunk(i, carry):
        base = base_w + i * K1_C
        pltpu.sync_copy(srcp.at[pl.ds(base, K1_C)], src_v)
        pltpu.sync_copy(dstp.at[pl.ds(base, K1_C)], dst_v)
        pltpu.async_copy(gmat.at[src_v], g_v, sem).wait()
        pltpu.async_copy(xmat.at[dst_v], x_v, sem).wait()
        pltpu.async_copy(umat.at[src_v], u_v, sem).wait()
        pltpu.async_copy(vmat.at[dst_v], v_v, sem).wait()

        def edge(e, carry2):
            acc_c = jnp.zeros((16,), jnp.float32)
            acc_g = jnp.zeros((16,), jnp.float32)
            for k in range(16):
                ks = pl.ds(k * 16, 16)
                acc_c += g_v[e, ks] * x_v[e, ks]
                z = u_v[e, ks] + v_v[e, ks]
                acc_g += tanh16(z) * ag_v[ks]
            part_v[pl.ds(e * 16, 16)] = acc_c
            part_v[pl.ds(K1_C * 16 + e * 16, 16)] = acc_g
            return carry2

        lax.fori_loop(0, K1_C, edge, 0, unroll=False)

        for g16 in range(K1_C // 16):
            rows = lax.iota(jnp.int32, 16) + g16 * 16
            s16 = src_v[pl.ds(g16 * 16, 16)]
            d16 = dst_v[pl.ds(g16 * 16, 16)]

            def sf(n16, f):
                return plsc.load_gather(tab_v, [n16 * 8 + f])

            lo_gat = lrelu(sf(s16, 0) + sf(d16, 1))
            lo_sym = (lrelu(sf(s16, 2) + sf(d16, 3))
                      + lrelu(sf(d16, 2) + sf(s16, 3)))
            lo_lin = tanh16(sf(s16, 4) + sf(d16, 5))
            lo_gen8 = lrelu(sf(s16, 6) + sf(d16, 7))
            lo_cos = jnp.zeros((16,), jnp.float32)
            lo_gnl = jnp.zeros((16,), jnp.float32)
            for k in range(16):
                lo_cos += plsc.load_gather(part_v, [rows * 16 + k])
                lo_gnl += plsc.load_gather(part_v, [K1_C * 16 + rows * 16 + k])
            for f, val in ((0, lo_gat), (1, lo_sym), (2, lo_cos),
                           (3, lo_lin), (4, lo_gnl), (5, lo_gen8)):
                plsc.store_scatter(lo_v, [rows * 16 + f], val)
        pltpu.sync_copy(lo_v, lo_out.at[pl.ds(base * 16, K1_C * 16)])
        return carry

    lax.fori_loop(0, epw // K1_C, chunk, 0, unroll=False)


def _k1_logits(srcp, dstp, sc8, gmat, xmat, umat, vmat, ag):
    epad = srcp.shape[0]
    npad = sc8.shape[0]
    mesh = plsc.VectorSubcoreMesh(core_axis_name="c", subcore_axis_name="s")
    f = pl.kernel(
        _k1_body,
        out_type=jax.ShapeDtypeStruct((epad * 16,), jnp.float32),
        mesh=mesh,
        compiler_params=pltpu.CompilerParams(needs_layout_passes=False),
        scratch_types=[
            pltpu.VMEM((K1_C,), jnp.int32),
            pltpu.VMEM((K1_C,), jnp.int32),
            pltpu.VMEM((npad * 8,), jnp.float32),
            pltpu.VMEM((K1_C, 256), jnp.float32),
            pltpu.VMEM((K1_C, 256), jnp.float32),
            pltpu.VMEM((K1_C, 256), jnp.float32),
            pltpu.VMEM((K1_C, 256), jnp.float32),
            pltpu.VMEM((256,), jnp.float32),
            pltpu.VMEM((K1_C * 16,), jnp.float32),
            pltpu.VMEM((2 * K1_C * 16,), jnp.float32),
            pltpu.SemaphoreType.DMA,
        ],
    )
    return f(srcp, dstp, sc8.reshape(-1), gmat, xmat, umat, vmat, ag)


K2_C = 128


def _k2_body(dstp, lo16f, maxpart, degpart,
             dst_v, lo_v, bins_v, dbins_v, sem):
    epw = dstp.shape[0] // SC_NW
    npad = degpart.shape[1]
    cid = lax.axis_index("c")
    sid = lax.axis_index("s")
    wid = sid * SC_NC + cid
    base_w = wid * epw
    neg = jnp.full((16,), -jnp.inf, jnp.float32)

    def binit(i, c):
        bins_v[pl.ds(i * 16, 16)] = neg
        return c

    lax.fori_loop(0, npad * 8 // 16, binit, 0, unroll=False)

    def dinit(i, c):
        dbins_v[pl.ds(i * 16, 16)] = jnp.zeros((16,), jnp.float32)
        return c

    lax.fori_loop(0, (npad + 16) // 16, dinit, 0, unroll=False)
    q16 = lax.iota(jnp.int32, 16)
    eoff = q16 >> 3
    fld = q16 & 7

    def chunk(i, carry):
        base = base_w + i * K2_C
        pltpu.sync_copy(dstp.at[pl.ds(base, K2_C)], dst_v)
        pltpu.sync_copy(lo16f.at[pl.ds(base * 16, K2_C * 16)], lo_v)

        def pair(p, c2):
            e_ids = p * 2 + eoff
            esw = e_ids ^ 1
            d16 = plsc.load_gather(dst_v, [e_ids])
            dsw = plsc.load_gather(dst_v, [esw])
            lov = plsc.load_gather(lo_v, [e_ids * 16 + fld])
            losw = plsc.load_gather(lo_v, [esw * 16 + fld])
            val = jnp.where(d16 == dsw, jnp.maximum(lov, losw), lov)
            idx = d16 * 8 + fld
            cur = plsc.load_gather(bins_v, [idx])
            plsc.store_scatter(bins_v, [idx], jnp.maximum(cur, val))
            dval = jnp.where(d16 == dsw, 2.0, 1.0)
            dval = jnp.where(fld == 0, dval, 0.0)
            didx = jnp.where(fld == 0, d16, npad + q16)
            dcur = plsc.load_gather(dbins_v, [didx])
            plsc.store_scatter(dbins_v, [didx], dcur + dval)
            return c2

        lax.fori_loop(0, K2_C // 2, pair, 0, unroll=False)
        return carry

    lax.fori_loop(0, epw // K2_C, chunk, 0, unroll=False)
    pltpu.sync_copy(bins_v, maxpart.at[wid])
    pltpu.sync_copy(dbins_v.at[pl.ds(0, npad)], degpart.at[wid])


def _k2_maxdeg(dstp, lo16f, npad):
    epad = dstp.shape[0]
    mesh = plsc.VectorSubcoreMesh(core_axis_name="c", subcore_axis_name="s")
    f = pl.kernel(
        _k2_body,
        out_type=[jax.ShapeDtypeStruct((SC_NW, npad * 8), jnp.float32),
                  jax.ShapeDtypeStruct((SC_NW, npad), jnp.float32)],
        mesh=mesh,
        compiler_params=pltpu.CompilerParams(needs_layout_passes=False),
        scratch_types=[
            pltpu.VMEM((K2_C,), jnp.int32),
            pltpu.VMEM((K2_C * 16,), jnp.float32),
            pltpu.VMEM((npad * 8,), jnp.float32),
            pltpu.VMEM((npad + 16,), jnp.float32),
            pltpu.SemaphoreType.DMA,
        ],
    )
    return f(dstp, lo16f)


K3_C = 128


def _k3a_body(srcp, dstp, lo16f, mtab8f, numf_out,
              src_v, dst_v, lo_v, numf_v, mtab_v, sem):
    epw = srcp.shape[0] // SC_NW
    wid = lax.axis_index("s") * SC_NC + lax.axis_index("c")
    base_w = wid * epw
    pltpu.sync_copy(mtab8f, mtab_v)
    q16 = lax.iota(jnp.int32, 16)
    f8 = q16 & 7

    def chunk(i, carry):
        base = base_w + i * K3_C
        pltpu.sync_copy(srcp.at[pl.ds(base, K3_C)], src_v)
        pltpu.sync_copy(dstp.at[pl.ds(base, K3_C)], dst_v)
        pltpu.sync_copy(lo16f.at[pl.ds(base * 16, K3_C * 16)], lo_v)

        def edge(e, c2):
            e16 = jnp.full((16,), e, jnp.int32)
            d16 = plsc.load_gather(dst_v, [e16])
            s16 = plsc.load_gather(src_v, [e16])
            m16 = plsc.load_gather(mtab_v, [d16 * 8 + f8])
            dinv_d = plsc.load_gather(mtab_v, [d16 * 8 + 6])
            dinv_s = plsc.load_gather(mtab_v, [s16 * 8 + 6])
            lo16 = lo_v[pl.ds(e * 16, 16)]
            num16 = jnp.exp(lo16 - m16)
            num16 = jnp.where(q16 == 6, dinv_s * dinv_d, num16)
            num16 = jnp.where(q16 < 8, num16, 0.0)
            numf_v[pl.ds(e * 16, 16)] = num16
            return c2

        lax.fori_loop(0, K3_C, edge, 0, unroll=False)
        pltpu.sync_copy(numf_v, numf_out.at[pl.ds(base * 16, K3_C * 16)])
        return carry

    lax.fori_loop(0, epw // K3_C, chunk, 0, unroll=False)


def _k3b_body(dstp, numf, denpart, dst_v, numf_v, bins_v, sem):
    epw = dstp.shape[0] // SC_NW
    npad = denpart.shape[1] // 8
    wid = lax.axis_index("s") * SC_NC + lax.axis_index("c")
    base_w = wid * epw

    def binit(i, c):
        bins_v[pl.ds(i * 16, 16)] = jnp.zeros((16,), jnp.float32)
        return c

    lax.fori_loop(0, npad * 8 // 16, binit, 0, unroll=False)
    q16 = lax.iota(jnp.int32, 16)
    eoff = q16 >> 3
    fld = q16 & 7

    def chunk(i, carry):
        base = base_w + i * K3_C
        pltpu.sync_copy(dstp.at[pl.ds(base, K3_C)], dst_v)
        pltpu.sync_copy(numf.at[pl.ds(base * 16, K3_C * 16)], numf_v)

        def pair(p, c2):
            e_ids = p * 2 + eoff
            esw = e_ids ^ 1
            d16 = plsc.load_gather(dst_v, [e_ids])
            dsw = plsc.load_gather(dst_v, [esw])
            nv = plsc.load_gather(numf_v, [e_ids * 16 + fld])
            nsw = plsc.load_gather(numf_v, [esw * 16 + fld])
            val = jnp.where(d16 == dsw, nv + nsw, nv)
            idx = d16 * 8 + fld
            cur = plsc.load_gather(bins_v, [idx])
            plsc.store_scatter(bins_v, [idx], cur + val)
            return c2

        lax.fori_loop(0, K3_C // 2, pair, 0, unroll=False)
        return carry

    lax.fori_loop(0, epw // K3_C, chunk, 0, unroll=False)
    pltpu.sync_copy(bins_v, denpart.at[wid])


def _k3_num(srcp, dstp, lo16f, mtab8, npad):
    epad = srcp.shape[0]
    mesh = plsc.VectorSubcoreMesh(core_axis_name="c", subcore_axis_name="s")
    fa = pl.kernel(
        _k3a_body,
        out_type=jax.ShapeDtypeStruct((epad * 16,), jnp.float32),
        mesh=mesh,
        compiler_params=pltpu.CompilerParams(needs_layout_passes=False),
        scratch_types=[
            pltpu.VMEM((K3_C,), jnp.int32),
            pltpu.VMEM((K3_C,), jnp.int32),
            pltpu.VMEM((K3_C * 16,), jnp.float32),
            pltpu.VMEM((K3_C * 16,), jnp.float32),
            pltpu.VMEM((npad * 8,), jnp.float32),
            pltpu.SemaphoreType.DMA,
        ],
    )
    numf = fa(srcp, dstp, lo16f, mtab8.reshape(-1))
    fb = pl.kernel(
        _k3b_body,
        out_type=jax.ShapeDtypeStruct((SC_NW, npad * 8), jnp.float32),
        mesh=mesh,
        compiler_params=pltpu.CompilerParams(needs_layout_passes=False),
        scratch_types=[
            pltpu.VMEM((K3_C,), jnp.int32),
            pltpu.VMEM((K3_C * 16,), jnp.float32),
            pltpu.VMEM((npad * 8,), jnp.float32),
            pltpu.SemaphoreType.DMA,
        ],
    )
    denpart = fb(dstp, numf)
    return numf, denpart


def _mid2_body(denpart_ref, stab_ref):
    den = jnp.sum(denpart_ref[...], axis=0)
    inv = 1.0 / (den + 1e-16)
    q = lax.broadcasted_iota(jnp.int32, inv.shape, 1)
    stab_ref[...] = jnp.where(q < 6, inv, 1.0)


def _mid2(denpart, npad):
    nb = npad // N_PAD_BLK
    denp = denpart.reshape(SC_NW, npad, 8)
    return pl.pallas_call(
        _mid2_body,
        grid=(nb,),
        in_specs=[pl.BlockSpec((SC_NW, N_PAD_BLK, 8), lambda i: (0, i, 0))],
        out_specs=pl.BlockSpec((N_PAD_BLK, 8), lambda i: (i, 0)),
        out_shape=jax.ShapeDtypeStruct((npad, 8), jnp.float32),
    )(denp)


def _mid_body(maxpart_ref, degpart_ref, mtab_ref, invdeg_ref, dinv2_ref):
    m = jnp.max(maxpart_ref[...], axis=0)
    m = jnp.where(jnp.isfinite(m), m, 0.0)
    deg = jnp.sum(degpart_ref[...], axis=0)
    dinv = 1.0 / jnp.sqrt(deg + 1.0)
    blk = m.shape[0]
    mtab_ref[...] = jnp.concatenate(
        [m[:, :6], dinv[:, None], jnp.zeros((blk, 1), jnp.float32)], axis=1)
    invdeg_ref[...] = (1.0 / jnp.clip(deg, 1.0))[:, None]
    dinv2_ref[...] = (dinv * dinv)[:, None]


def _mid(maxpart, degpart, npad):
    nb = npad // N_PAD_BLK
    maxp = maxpart.reshape(SC_NW, npad, 8)
    return pl.pallas_call(
        _mid_body,
        grid=(nb,),
        in_specs=[pl.BlockSpec((SC_NW, N_PAD_BLK, 8), lambda i: (0, i, 0)),
                  pl.BlockSpec((SC_NW, N_PAD_BLK), lambda i: (0, i))],
        out_specs=[pl.BlockSpec((N_PAD_BLK, 8), lambda i: (i, 0)),
                   pl.BlockSpec((N_PAD_BLK, 1), lambda i: (i, 0)),
                   pl.BlockSpec((N_PAD_BLK, 1), lambda i: (i, 0))],
        out_shape=[jax.ShapeDtypeStruct((npad, 8), jnp.float32),
                   jax.ShapeDtypeStruct((npad, 1), jnp.float32),
                   jax.ShapeDtypeStruct((npad, 1), jnp.float32)],
    )(maxp, degpart)


def _k3c_body(dstp, numf, stab8f, wf_out, dst_v, numf_v, w_v, stab_v, sem):
    epw = dstp.shape[0] // SC_NW
    wid = lax.axis_index("s") * SC_NC + lax.axis_index("c")
    base_w = wid * epw
    pltpu.sync_copy(stab8f, stab_v)

    def zinit(i, c):
        w_v[pl.ds(i * 16, 16)] = jnp.zeros((16,), jnp.float32)
        return c

    lax.fori_loop(0, K3_C, zinit, 0, unroll=False)
    q16 = lax.iota(jnp.int32, 16)
    eoff = q16 >> 3
    fld = q16 & 7

    def chunk(i, carry):
        base = base_w + i * K3_C
        pltpu.sync_copy(dstp.at[pl.ds(base, K3_C)], dst_v)
        pltpu.sync_copy(numf.at[pl.ds(base * 16, K3_C * 16)], numf_v)

        def pair(p, c2):
            e_ids = p * 2 + eoff
            d16 = plsc.load_gather(dst_v, [e_ids])
            nv = plsc.load_gather(numf_v, [e_ids * 16 + fld])
            sv = plsc.load_gather(stab_v, [d16 * 8 + fld])
            plsc.store_scatter(w_v, [e_ids * 16 + fld], nv * sv)
            return c2

        lax.fori_loop(0, K3_C // 2, pair, 0, unroll=False)
        pltpu.sync_copy(w_v, wf_out.at[pl.ds(base * 16, K3_C * 16)])
        return carry

    lax.fori_loop(0, epw // K3_C, chunk, 0, unroll=False)


def _k3c_w(dstp, numf, stab8, npad):
    epad = dstp.shape[0]
    mesh = plsc.VectorSubcoreMesh(core_axis_name="c", subcore_axis_name="s")
    f = pl.kernel(
        _k3c_body,
        out_type=jax.ShapeDtypeStruct((epad * 16,), jnp.float32),
        mesh=mesh,
        compiler_params=pltpu.CompilerParams(needs_layout_passes=False),
        scratch_types=[
            pltpu.VMEM((K3_C,), jnp.int32),
            pltpu.VMEM((K3_C * 16,), jnp.float32),
            pltpu.VMEM((K3_C * 16,), jnp.float32),
            pltpu.VMEM((npad * 8,), jnp.float32),
            pltpu.SemaphoreType.DMA,
        ],
    )
    return f(dstp, numf, stab8.reshape(-1))


K5_C = 128


def _k5_body(srcp, dstp, wf, xh2, apart,
             src_v, idx_v, dst_v, w_v, x_v, out_v, z_v, acc, sem):
    epad = srcp.shape[0]
    epw = epad // SC_NS
    npad = apart.shape[2]
    cid = lax.axis_index("c")
    sid = lax.axis_index("s")
    base_w = sid * epw
    off = cid * npad
    nseg = npad // SC_NS

    def zrow(r, c):
        for t in range(8):
            z_v[r, pl.ds(t * 16, 16)] = jnp.zeros((16,), jnp.float32)
        return c

    lax.fori_loop(0, 16, zrow, 0, unroll=False)

    def pbody(p, carry):
        def zseg(t, c):
            pltpu.sync_copy(z_v, acc.at[pl.ds(sid * nseg + t * 16, 16)])
            return c

        lax.fori_loop(0, nseg // 16, zseg, 0, unroll=False)
        plsc.subcore_barrier()

        def chunk(i, carry2):
            base = base_w + i * K5_C
            pltpu.sync_copy(srcp.at[pl.ds(base, K5_C)], src_v)
            pltpu.sync_copy(dstp.at[pl.ds(base, K5_C)], dst_v)
            pltpu.sync_copy(wf.at[pl.ds(base * 16, K5_C * 16)], w_v)

            def adj(j, c2):
                idx_v[pl.ds(j * 16, 16)] = src_v[pl.ds(j * 16, 16)] + off
                return c2

            lax.fori_loop(0, K5_C // 16, adj, 0, unroll=False)
            pltpu.async_copy(xh2.at[idx_v], x_v, sem).wait()

            def edge(e, c2):
                wsplat = plsc.load_gather(
                    w_v, [jnp.full((16,), e * 16 + p, jnp.int32)])
                for t in range(8):
                    ts = pl.ds(t * 16, 16)
                    out_v[e, ts] = wsplat * x_v[e, ts]
                return c2

            lax.fori_loop(0, K5_C, edge, 0, unroll=False)
            pltpu.sync_copy(out_v, acc.at[dst_v], add=True)
            return carry2

        lax.fori_loop(0, epw // K5_C, chunk, 0, unroll=False)
        plsc.subcore_barrier()
        pltpu.sync_copy(acc.at[pl.ds(sid * nseg, nseg)],
                        apart.at[cid, p, pl.ds(sid * nseg, nseg)])
        plsc.subcore_barrier()
        return carry

    lax.fori_loop(0, 8, pbody, 0, unroll=False)


def _k5_accs(srcp, dstp, wf, xp, npad):
    epad = srcp.shape[0]
    xh2 = jnp.concatenate([xp[:, :128], xp[:, 128:]], axis=0)
    mesh = plsc.VectorSubcoreMesh(core_axis_name="c", subcore_axis_name="s")
    f = pl.kernel(
        _k5_body,
        out_type=jax.ShapeDtypeStruct((SC_NC, 8, npad, 128), jnp.float32),
        mesh=mesh,
        compiler_params=pltpu.CompilerParams(needs_layout_passes=False),
        scratch_types=[
            pltpu.VMEM((K5_C,), jnp.int32),
            pltpu.VMEM((K5_C,), jnp.int32),
            pltpu.VMEM((K5_C,), jnp.int32),
            pltpu.VMEM((K5_C * 16,), jnp.float32),
            pltpu.VMEM((K5_C, 128), jnp.float32),
            pltpu.VMEM((K5_C, 128), jnp.float32),
            pltpu.VMEM((16, 128), jnp.float32),
            pltpu.VMEM_SHARED((npad, 128), jnp.float32),
            pltpu.SemaphoreType.DMA,
        ],
    )
    apart = f(srcp, dstp, wf, xh2)
    return jnp.concatenate([apart[0], apart[1]], axis=-1)


def kernel(x, weights, edge_index, params):
    n = x.shape[0]
    npad = ((n + N_PAD_BLK - 1) // N_PAD_BLK) * N_PAD_BLK
    xp = jnp.pad(x, ((0, npad - n), (0, 0)))
    src = edge_index[0]
    dst = edge_index[1]

    e = src.shape[0]
    epad = ((e + 8 * SC_NW * K1_C - 1) // (8 * SC_NW * K1_C)) * (8 * SC_NW * K1_C)
    srcp = jnp.concatenate([src, jnp.zeros((epad - e,), jnp.int32)])
    dstp = jnp.concatenate([dst, jnp.full((epad - e,), npad - 1, jnp.int32)])

    wbig = _weight_prep(params)
    sc8, G, U, V = _node_prep(xp, wbig)
    ag = params['gat_generalized_linear']['ag']
    lo16f = _k1_logits(srcp, dstp, sc8, G, xp, U, V, ag)

    maxpart, degpart = _k2_maxdeg(dstp, lo16f, npad)
    mtab, invdeg, dinv2 = _mid(maxpart, degpart, npad)

    numf, denpart = _k3_num(srcp, dstp, lo16f, mtab, npad)
    stab = _mid2(denpart, npad)
    wf = _k3c_w(dstp, numf, stab, npad)
    a8 = _k5_accs(srcp, dstp, wf, xp, npad)
    aatt = a8[:6]
    agcn = a8[6]
    a1 = a8[7]

    xs = xp[src]
    amax = jax.ops.segment_max(xs, dst, num_segments=npad)
    amax = jnp.where(jnp.isfinite(amax), amax, 0.0)

    res = _final(xp, a1, amax, agcn, aatt, invdeg, dinv2, weights, params)
    return res[:n]
